# Initial kernel scaffold; baseline (speedup 1.0000x reference)
import jax
import jax.numpy as jnp
from jax.experimental import pallas as pl


def kernel(resids_positional_encoded, weight):
    raise NotImplementedError("write your pallas kernel here")


# speedup vs baseline: 4.9475x; 4.9475x over previous
import functools

import jax
import jax.numpy as jnp
from jax import lax
from jax.experimental import pallas as pl
from jax.experimental.pallas import tpu as pltpu
from jax.experimental.pallas import tpu_sc as plsc

_NUM_CORES = 2
_NUM_SUBCORES = 16
_NUM_WORKERS = _NUM_CORES * _NUM_SUBCORES
_CHUNK = 2048


@functools.cache
def _build_gather(n, d):
    per_w = n // _NUM_WORKERS
    nchunks = per_w // _CHUNK
    assert per_w * _NUM_WORKERS == n and nchunks * _CHUNK == per_w

    mesh = plsc.VectorSubcoreMesh(core_axis_name="c", subcore_axis_name="s")

    @functools.partial(
        pl.kernel,
        out_type=jax.ShapeDtypeStruct((n, d), jnp.float32),
        mesh=mesh,
        scratch_types=[
            pltpu.VMEM((_CHUNK,), jnp.int32),
            pltpu.VMEM((_CHUNK, d), jnp.float32),
            pltpu.SemaphoreType.DMA,
        ],
        compiler_params=pltpu.CompilerParams(use_tc_tiling_on_sc=False),
    )
    def gather_kernel(table_hbm, idx_hbm, out_hbm, idx_v, rows_v, sem):
        wid = lax.axis_index("s") * _NUM_CORES + lax.axis_index("c")
        base0 = wid * per_w

        def step(i, carry):
            base = base0 + i * _CHUNK
            pltpu.sync_copy(idx_hbm.at[pl.ds(base, _CHUNK)], idx_v)
            pltpu.async_copy(table_hbm.at[idx_v], rows_v, sem).wait()
            pltpu.sync_copy(rows_v, out_hbm.at[pl.ds(base, _CHUNK)])
            return carry

        lax.fori_loop(0, nchunks, step, 0)

    return gather_kernel


def kernel(resids_positional_encoded, weight):
    b, h = resids_positional_encoded.shape
    _, d = weight.shape
    idx = resids_positional_encoded.reshape(-1).astype(jnp.int32)
    out = _build_gather(idx.shape[0], d)(weight, idx)
    return out.reshape(b, h, d)


# speedup vs baseline: 4.9834x; 1.0073x over previous
import functools

import jax
import jax.numpy as jnp
from jax import lax
from jax.experimental import pallas as pl
from jax.experimental.pallas import tpu as pltpu
from jax.experimental.pallas import tpu_sc as plsc

_NUM_CORES = 2
_NUM_SUBCORES = 16
_NUM_WORKERS = _NUM_CORES * _NUM_SUBCORES
_CHUNK = 1600


@functools.cache
def _build_gather(n, d):
    per_w = n // _NUM_WORKERS
    nchunks = per_w // _CHUNK
    assert per_w * _NUM_WORKERS == n and nchunks * _CHUNK == per_w
    assert nchunks >= 2 and nchunks % 2 == 0

    mesh = plsc.VectorSubcoreMesh(core_axis_name="c", subcore_axis_name="s")

    @functools.partial(
        pl.kernel,
        out_type=jax.ShapeDtypeStruct((n, d), jnp.float32),
        mesh=mesh,
        scratch_types=[
            pltpu.VMEM((2, _CHUNK), jnp.int32),
            pltpu.VMEM((2, _CHUNK, d), jnp.float32),
            pltpu.SemaphoreType.DMA,
            pltpu.SemaphoreType.DMA,
            pltpu.SemaphoreType.DMA,
            pltpu.SemaphoreType.DMA,
        ],
        compiler_params=pltpu.CompilerParams(use_tc_tiling_on_sc=False),
    )
    def gather_kernel(table_hbm, idx_hbm, out_hbm, idx_v, rows_v, gs0, gs1, ss0, ss1):
        wid = lax.axis_index("s") * _NUM_CORES + lax.axis_index("c")
        base0 = wid * per_w
        gsem = [gs0, gs1]
        ssem = [ss0, ss1]

        def fire_gather(i, b):
            base = base0 + i * _CHUNK
            pltpu.sync_copy(idx_hbm.at[pl.ds(base, _CHUNK)], idx_v.at[b])
            pltpu.async_copy(table_hbm.at[idx_v.at[b]], rows_v.at[b], gsem[b])

        def wait_gather(b):
            pltpu.make_async_copy(
                table_hbm.at[idx_v.at[b]], rows_v.at[b], gsem[b]
            ).wait()

        def fire_store(i, b):
            base = base0 + i * _CHUNK
            pltpu.async_copy(rows_v.at[b], out_hbm.at[pl.ds(base, _CHUNK)], ssem[b])

        def wait_store(b):
            pltpu.make_async_copy(
                rows_v.at[b], out_hbm.at[pl.ds(base0, _CHUNK)], ssem[b]
            ).wait()

        fire_gather(0, 0)

        def chunk_step(i, b):
            @pl.when(i >= 1)
            def _():
                wait_store(1 - b)

            @pl.when(i + 1 < nchunks)
            def _():
                fire_gather(i + 1, 1 - b)

            wait_gather(b)
            fire_store(i, b)

        def body(k, carry):
            for b in range(2):
                chunk_step(2 * k + b, b)
            return carry

        lax.fori_loop(0, nchunks // 2, body, 0, unroll=False)
        wait_store(1)

    return gather_kernel


def kernel(resids_positional_encoded, weight):
    b, h = resids_positional_encoded.shape
    _, d = weight.shape
    idx = resids_positional_encoded.reshape(-1).astype(jnp.int32)
    out = _build_gather(idx.shape[0], d)(weight, idx)
    return out.reshape(b, h, d)
